# baseline (speedup 1.0000x reference)
import functools

import jax
import jax.numpy as jnp
from jax import lax
from jax.experimental import pallas as pl
from jax.experimental.pallas import tpu as pltpu
from jax.experimental.pallas import tpu_sc as plsc

B, S = 4, 8192
N = B * S
VOCAB, HID, OUT = 32, 16, 32

_info = plsc.get_sparse_core_info()
_NC, _NS = _info.num_cores, _info.num_subcores
_NW = _NC * _NS
_BPW = N // _NW
_L = 16

_mesh = plsc.VectorSubcoreMesh(core_axis_name="c", subcore_axis_name="s")


@functools.partial(
    pl.kernel,
    mesh=_mesh,
    out_type=jax.ShapeDtypeStruct((_NW, _BPW * OUT), jnp.float32),
    scratch_types=[
        pltpu.VMEM((_BPW,), jnp.int32),
        pltpu.VMEM((VOCAB, OUT), jnp.float32),
        pltpu.VMEM((_BPW * OUT,), jnp.float32),
        pltpu.VMEM((VOCAB, HID), jnp.float32),
        pltpu.VMEM((HID, HID), jnp.float32),
        pltpu.VMEM((HID,), jnp.float32),
        pltpu.VMEM((HID, OUT), jnp.float32),
        pltpu.VMEM((OUT,), jnp.float32),
        pltpu.SemaphoreType.DMA,
    ],
    compiler_params=pltpu.CompilerParams(
        use_tc_tiling_on_sc=False, needs_layout_passes=False
    ),
)
def _fused_call(
    ids_hbm, embed_hbm, wpt_hbm, bp_hbm, wht_hbm, bh_hbm, out_hbm,
    ids_v, table_v, rows_v, embed_v, wpt_v, bp_v, wht_v, bh_v, sem,
):
    wid = lax.axis_index("s") * _NC + lax.axis_index("c")
    ids_copy = pltpu.async_copy(ids_hbm.at[wid], ids_v, sem)
    pltpu.sync_copy(embed_hbm, embed_v)
    pltpu.sync_copy(wpt_hbm, wpt_v)
    pltpu.sync_copy(bp_hbm, bp_v)
    pltpu.sync_copy(wht_hbm, wht_v)
    pltpu.sync_copy(bh_hbm, bh_v)

    bp = bp_v[...]
    bh0 = bh_v[pl.ds(0, _L)]
    bh1 = bh_v[pl.ds(_L, _L)]

    def _tree_sum(terms):
        while len(terms) > 1:
            nxt = [terms[i] + terms[i + 1] for i in range(0, len(terms) - 1, 2)]
            if len(terms) % 2:
                nxt.append(terms[-1])
            terms = nxt
        return terms[0]

    @plsc.parallel_loop(0, VOCAB, unroll=4)
    def _vocab_row(v):
        e = embed_v[v, :]
        h = _tree_sum([e[k] * wpt_v[k, :] for k in range(HID)] + [bp])
        h = jnp.maximum(h, 0.0)
        hs = [h[i] for i in range(HID)]
        t0 = _tree_sum([hs[i] * wht_v[i, pl.ds(0, _L)] for i in range(HID)] + [bh0])
        t1 = _tree_sum([hs[i] * wht_v[i, pl.ds(_L, _L)] for i in range(HID)] + [bh1])
        table_v[v, pl.ds(0, _L)] = t0
        table_v[v, pl.ds(_L, _L)] = t1

    ids_copy.wait()
    iota = lax.iota(jnp.int32, _L)
    zeros = jnp.zeros((_L,), jnp.int32)

    @plsc.parallel_loop(0, _BPW // _L, unroll=2)
    def _group(g):
        ids16 = ids_v[pl.ds(g * _L, _L)]
        off = g * (_L * OUT)
        for l in range(_L):
            row = zeros + ids16[l]
            rows_v[pl.ds(off + l * OUT, _L)] = plsc.load_gather(
                table_v, [row, iota]
            )
            rows_v[pl.ds(off + l * OUT + _L, _L)] = plsc.load_gather(
                table_v, [row, iota + _L]
            )

    pltpu.sync_copy(rows_v, out_hbm.at[wid])


def kernel(input_ids, embed, W_proj, b_proj, W_head, b_head):
    ids = input_ids.reshape(_NW, _BPW)
    out = _fused_call(ids, embed, W_proj.T, b_proj, W_head.T, b_head)
    return out.reshape(B, S, OUT)


# speedup vs baseline: 1.0924x; 1.0924x over previous
import functools

import jax
import jax.numpy as jnp
from jax import lax
from jax.experimental import pallas as pl
from jax.experimental.pallas import tpu as pltpu
from jax.experimental.pallas import tpu_sc as plsc

B, S = 4, 8192
N = B * S
VOCAB, HID, OUT = 32, 16, 32

_info = plsc.get_sparse_core_info()
_NC, _NS = _info.num_cores, _info.num_subcores
_NW = _NC * _NS
_BPW = N // _NW
_L = 16

_mesh = plsc.VectorSubcoreMesh(core_axis_name="c", subcore_axis_name="s")


@functools.partial(
    pl.kernel,
    mesh=_mesh,
    out_type=jax.ShapeDtypeStruct((_NW, _BPW * OUT), jnp.float32),
    scratch_types=[
        pltpu.VMEM((_BPW,), jnp.int32),
        pltpu.VMEM((VOCAB, OUT), jnp.float32),
        pltpu.VMEM((_BPW * OUT,), jnp.float32),
        pltpu.VMEM((2, HID), jnp.float32),
        pltpu.VMEM((HID, HID), jnp.float32),
        pltpu.VMEM((HID,), jnp.float32),
        pltpu.VMEM((HID, OUT), jnp.float32),
        pltpu.VMEM((OUT,), jnp.float32),
        pltpu.VMEM((2, OUT), jnp.float32),
        pltpu.VMEM_SHARED((VOCAB, OUT), jnp.float32),
        pltpu.SemaphoreType.DMA,
        pltpu.SemaphoreType.DMA,
    ],
    compiler_params=pltpu.CompilerParams(
        use_tc_tiling_on_sc=False, needs_layout_passes=False
    ),
)
def _fused_call(
    ids_hbm, embed_hbm, wpt_hbm, bp_hbm, wht_hbm, bh_hbm, out_hbm,
    ids_v, table_v, rows_v, embed2_v, wpt_v, bp_v, wht_v, bh_v, row2_v,
    table_sh, sem_ids, sem_w,
):
    cid = lax.axis_index("c")
    sid = lax.axis_index("s")
    wid = sid * _NC + cid
    ids_copy = pltpu.async_copy(ids_hbm.at[wid], ids_v, sem_ids)
    w_copies = [
        pltpu.async_copy(embed_hbm.at[pl.ds(sid * 2, 2)], embed2_v, sem_w),
        pltpu.async_copy(wpt_hbm, wpt_v, sem_w),
        pltpu.async_copy(bp_hbm, bp_v, sem_w),
        pltpu.async_copy(wht_hbm, wht_v, sem_w),
        pltpu.async_copy(bh_hbm, bh_v, sem_w),
    ]
    for c in w_copies:
        c.wait()

    bp = bp_v[...]
    bh0 = bh_v[pl.ds(0, _L)]
    bh1 = bh_v[pl.ds(_L, _L)]

    def _tree_sum(terms):
        while len(terms) > 1:
            nxt = [terms[i] + terms[i + 1] for i in range(0, len(terms) - 1, 2)]
            if len(terms) % 2:
                nxt.append(terms[-1])
            terms = nxt
        return terms[0]

    for v in range(2):
        e = embed2_v[v, :]
        h = _tree_sum([e[k] * wpt_v[k, :] for k in range(HID)] + [bp])
        h = jnp.maximum(h, 0.0)
        hs = [h[i] for i in range(HID)]
        t0 = _tree_sum([hs[i] * wht_v[i, pl.ds(0, _L)] for i in range(HID)] + [bh0])
        t1 = _tree_sum([hs[i] * wht_v[i, pl.ds(_L, _L)] for i in range(HID)] + [bh1])
        row2_v[v, pl.ds(0, _L)] = t0
        row2_v[v, pl.ds(_L, _L)] = t1

    pltpu.sync_copy(row2_v, table_sh.at[pl.ds(sid * 2, 2)])
    plsc.subcore_barrier()
    pltpu.sync_copy(table_sh, table_v)

    ids_copy.wait()
    iota = lax.iota(jnp.int32, _L)
    zeros = jnp.zeros((_L,), jnp.int32)

    @plsc.parallel_loop(0, _BPW // _L, unroll=2)
    def _group(g):
        ids16 = ids_v[pl.ds(g * _L, _L)]
        off = g * (_L * OUT)
        for l in range(_L):
            row = zeros + ids16[l]
            rows_v[pl.ds(off + l * OUT, _L)] = plsc.load_gather(
                table_v, [row, iota]
            )
            rows_v[pl.ds(off + l * OUT + _L, _L)] = plsc.load_gather(
                table_v, [row, iota + _L]
            )

    pltpu.sync_copy(rows_v, out_hbm.at[wid])


def kernel(input_ids, embed, W_proj, b_proj, W_head, b_head):
    ids = input_ids.reshape(_NW, _BPW)
    out = _fused_call(ids, embed, W_proj.T, b_proj, W_head.T, b_head)
    return out.reshape(B, S, OUT)
